# Initial kernel scaffold; baseline (speedup 1.0000x reference)
import functools

import jax
import jax.numpy as jnp
from jax.experimental import pallas as pl
from jax.experimental.pallas import tpu as pltpu


def _dense_body(nE, x_ref, gw_ref, gb_ref, w1_ref, b1_ref, w2_ref, b2_ref,
                out_ref):
    e = pl.program_id(1)
    x = x_ref[...]
    logits = jax.lax.dot(x, gw_ref[...],
                         preferred_element_type=jnp.float32) + gb_ref[...]
    p = jax.nn.softmax(logits, axis=-1)
    idx = jax.lax.broadcasted_iota(jnp.int32, p.shape, 1)
    m0 = jnp.max(p, axis=-1, keepdims=True)
    e0 = jnp.min(jnp.where(p == m0, idx, nE), axis=-1, keepdims=True)
    p2 = jnp.where(idx == e0, -jnp.inf, p)
    m1 = jnp.max(p2, axis=-1, keepdims=True)
    e1 = jnp.min(jnp.where(p2 == m1, idx, nE), axis=-1, keepdims=True)
    w_e = (jnp.where(e0[:, 0] == e, m0[:, 0], 0.0)
           + jnp.where(e1[:, 0] == e, m1[:, 0], 0.0))
    h = jax.lax.dot(x, w1_ref[0], preferred_element_type=jnp.float32)
    h = h + b1_ref[...]
    h = jax.nn.gelu(h, approximate=False)
    y = jax.lax.dot(h, w2_ref[0], preferred_element_type=jnp.float32)
    y = y + b2_ref[...]

    @pl.when(e == 0)
    def _():
        out_ref[...] = jnp.zeros_like(out_ref)

    out_ref[...] += w_e[:, None] * y


def kernel(x, gate_W, gate_b, W1, b1, W2, b2):
    B, L, D = x.shape
    E = gate_W.shape[1]
    F = W1.shape[2]
    N = B * L
    T = 1024
    nt = N // T
    xf = x.reshape(N, D)
    gb2 = gate_b.reshape(1, E)

    out = pl.pallas_call(
        functools.partial(_dense_body, E),
        grid=(nt, E),
        in_specs=[
            pl.BlockSpec((T, D), lambda i, e: (i, 0)),
            pl.BlockSpec((D, E), lambda i, e: (0, 0)),
            pl.BlockSpec((1, E), lambda i, e: (0, 0)),
            pl.BlockSpec((1, D, F), lambda i, e: (e, 0, 0)),
            pl.BlockSpec((1, F), lambda i, e: (e, 0)),
            pl.BlockSpec((1, F, D), lambda i, e: (e, 0, 0)),
            pl.BlockSpec((1, D), lambda i, e: (e, 0)),
        ],
        out_specs=pl.BlockSpec((T, D), lambda i, e: (i, 0)),
        out_shape=jax.ShapeDtypeStruct((N, D), jnp.float32),
        compiler_params=pltpu.CompilerParams(
            dimension_semantics=("arbitrary", "arbitrary"),
        ),
    )(xf, gate_W, gb2, W1, b1, W2, b2)
    return out.reshape(B, L, D)


# speedup vs baseline: 2.9729x; 2.9729x over previous
import functools

import jax
import jax.numpy as jnp
from jax.experimental import pallas as pl
from jax.experimental.pallas import tpu as pltpu


def _dense_body(nE, x_ref, gw_ref, gb_ref, w1_ref, b1_ref, w2_ref, b2_ref,
                out_ref):
    e = pl.program_id(1)
    x = x_ref[...]
    logits = jax.lax.dot(x, gw_ref[...],
                         preferred_element_type=jnp.float32) + gb_ref[...]
    p = jax.nn.softmax(logits, axis=-1)
    idx = jax.lax.broadcasted_iota(jnp.int32, p.shape, 1)
    m0 = jnp.max(p, axis=-1, keepdims=True)
    e0 = jnp.min(jnp.where(p == m0, idx, nE), axis=-1, keepdims=True)
    p2 = jnp.where(idx == e0, -jnp.inf, p)
    m1 = jnp.max(p2, axis=-1, keepdims=True)
    e1 = jnp.min(jnp.where(p2 == m1, idx, nE), axis=-1, keepdims=True)
    w_e = (jnp.where(e0[:, 0] == e, m0[:, 0], 0.0)
           + jnp.where(e1[:, 0] == e, m1[:, 0], 0.0))
    h = jax.lax.dot(x, w1_ref[0], preferred_element_type=jnp.float32)
    h = h + b1_ref[0]
    h = 0.5 * h * (1.0 + jax.lax.erf(h * (2.0 ** -0.5)))
    y = jax.lax.dot(h, w2_ref[0], preferred_element_type=jnp.float32)
    y = y + b2_ref[0]

    @pl.when(e == 0)
    def _():
        out_ref[...] = jnp.zeros_like(out_ref)

    out_ref[...] += w_e[:, None] * y


def kernel(x, gate_W, gate_b, W1, b1, W2, b2):
    B, L, D = x.shape
    E = gate_W.shape[1]
    F = W1.shape[2]
    N = B * L
    T = min(512, N)
    nt = N // T
    xf = x.reshape(N, D)
    gb2 = gate_b.reshape(1, E)
    b1r = b1.reshape(E, 1, F)
    b2r = b2.reshape(E, 1, D)

    out = pl.pallas_call(
        functools.partial(_dense_body, E),
        grid=(nt, E),
        in_specs=[
            pl.BlockSpec((T, D), lambda i, e: (i, 0)),
            pl.BlockSpec((D, E), lambda i, e: (0, 0)),
            pl.BlockSpec((1, E), lambda i, e: (0, 0)),
            pl.BlockSpec((1, D, F), lambda i, e: (e, 0, 0)),
            pl.BlockSpec((1, 1, F), lambda i, e: (e, 0, 0)),
            pl.BlockSpec((1, F, D), lambda i, e: (e, 0, 0)),
            pl.BlockSpec((1, 1, D), lambda i, e: (e, 0, 0)),
        ],
        out_specs=pl.BlockSpec((T, D), lambda i, e: (i, 0)),
        out_shape=jax.ShapeDtypeStruct((N, D), jnp.float32),
        compiler_params=pltpu.CompilerParams(
            dimension_semantics=("arbitrary", "arbitrary"),
        ),
    )(xf, gate_W, gb2, W1, b1r, W2, b2r)
    return out.reshape(B, L, D)


# speedup vs baseline: 3.3252x; 1.1185x over previous
import functools

import jax
import jax.numpy as jnp
from jax.experimental import pallas as pl
from jax.experimental.pallas import tpu as pltpu


def _gate_body(nE, x_ref, gw_ref, gb_ref, w01_ref, e01_ref):
    x = x_ref[...]
    logits = jax.lax.dot(x, gw_ref[...],
                         preferred_element_type=jnp.float32) + gb_ref[...]
    p = jax.nn.softmax(logits, axis=-1)
    idx = jax.lax.broadcasted_iota(jnp.int32, p.shape, 1)
    m0 = jnp.max(p, axis=-1, keepdims=True)
    e0 = jnp.min(jnp.where(p == m0, idx, nE), axis=-1, keepdims=True)
    p2 = jnp.where(idx == e0, -jnp.inf, p)
    m1 = jnp.max(p2, axis=-1, keepdims=True)
    e1 = jnp.min(jnp.where(p2 == m1, idx, nE), axis=-1, keepdims=True)
    w01_ref[0, :] = m0[:, 0]
    w01_ref[1, :] = m1[:, 0]
    e01_ref[0, :] = e0[:, 0]
    e01_ref[1, :] = e1[:, 0]


def _gate(xf, gate_W, gate_b):
    N, D = xf.shape
    E = gate_W.shape[1]
    return pl.pallas_call(
        functools.partial(_gate_body, E),
        in_specs=[
            pl.BlockSpec((N, D), lambda: (0, 0)),
            pl.BlockSpec((D, E), lambda: (0, 0)),
            pl.BlockSpec((1, E), lambda: (0, 0)),
        ],
        out_specs=[
            pl.BlockSpec((2, N), lambda: (0, 0)),
            pl.BlockSpec((2, N), lambda: (0, 0)),
        ],
        out_shape=[
            jax.ShapeDtypeStruct((2, N), jnp.float32),
            jax.ShapeDtypeStruct((2, N), jnp.int32),
        ],
    )(xf, gate_W, gate_b.reshape(1, E))


def _ffn_body(ept_ref, x_ref, w1_ref, b1_ref, w2_ref, b2_ref, y_ref):
    x = x_ref[...]
    h = jax.lax.dot(x, w1_ref[0], preferred_element_type=jnp.float32)
    h = h + b1_ref[0]
    h = 0.5 * h * (1.0 + jax.lax.erf(h * (2.0 ** -0.5)))
    y = jax.lax.dot(h, w2_ref[0], preferred_element_type=jnp.float32)
    y_ref[...] = y + b2_ref[0]


def _grouped_ffn(x_sorted, ept, W1, b1, W2, b2, T):
    Amax, D = x_sorted.shape
    E, _, F = W1.shape
    nt = Amax // T
    grid_spec = pltpu.PrefetchScalarGridSpec(
        num_scalar_prefetch=1,
        grid=(nt,),
        in_specs=[
            pl.BlockSpec((T, D), lambda i, ept: (i, 0)),
            pl.BlockSpec((1, D, F), lambda i, ept: (ept[i], 0, 0)),
            pl.BlockSpec((1, 1, F), lambda i, ept: (ept[i], 0, 0)),
            pl.BlockSpec((1, F, D), lambda i, ept: (ept[i], 0, 0)),
            pl.BlockSpec((1, 1, D), lambda i, ept: (ept[i], 0, 0)),
        ],
        out_specs=pl.BlockSpec((T, D), lambda i, ept: (i, 0)),
    )
    return pl.pallas_call(
        _ffn_body,
        grid_spec=grid_spec,
        out_shape=jax.ShapeDtypeStruct((Amax, D), jnp.float32),
        compiler_params=pltpu.CompilerParams(
            dimension_semantics=("arbitrary",),
        ),
    )(ept, x_sorted, W1, b1.reshape(E, 1, F), W2, b2.reshape(E, 1, D))


def kernel(x, gate_W, gate_b, W1, b1, W2, b2):
    B, L, D = x.shape
    E = gate_W.shape[1]
    F = W1.shape[2]
    N = B * L
    A = 2 * N
    T = 256
    Amax = A + 8 * T
    nt = Amax // T

    xf = x.reshape(N, D)
    w01, e01 = _gate(xf, gate_W, gate_b)

    e_a = e01.reshape(A)
    counts = jnp.sum(e_a[:, None] == jnp.arange(E)[None, :], axis=0)
    padded = ((counts + T - 1) // T) * T
    ends = jnp.cumsum(padded)
    seg_start = ends - padded
    order = jnp.argsort(e_a, stable=True)
    e_sorted = e_a[order]
    cum_excl = jnp.cumsum(counts) - counts
    dst_sorted = seg_start[e_sorted] + (jnp.arange(A) - cum_excl[e_sorted])
    dst = jnp.zeros((A,), jnp.int32).at[order].set(dst_sorted.astype(jnp.int32))
    tile_start = jnp.arange(nt) * T
    ept = jnp.minimum(
        jnp.sum(tile_start[:, None] >= ends[None, :], axis=1), E - 1
    ).astype(jnp.int32)

    src_tok = jnp.zeros((Amax,), jnp.int32).at[dst].set(
        jnp.arange(A, dtype=jnp.int32) % N)
    x_sorted = xf[src_tok]

    y_sorted = _grouped_ffn(x_sorted, ept, W1, b1, W2, b2, T)

    out = (w01[0][:, None] * y_sorted[dst[:N]]
           + w01[1][:, None] * y_sorted[dst[N:]])
    return out.reshape(B, L, D)
